# baseline (speedup 1.0000x reference)
import jax
import jax.numpy as jnp
from jax import lax
from jax.experimental import pallas as pl
from jax.experimental.pallas import tpu as pltpu
from jax.experimental.pallas import tpu_sc as plsc

B, N, M, C = 16, 8400, 64, 80
TOPK = 10
L = 16
NP = 8704
NK = 512
NCHUNK = NP // NK
VPC = NK // L
MH = M // 2
HN0 = 4208
HN1 = N - HN0
PN = NP
MRG = 4224
SL = M * L


def _sc_body(srows, pdtflat, ancflat, gtall, out_o,
             rows_v, basebuf, cidx, x1c, y1c, x2c, y2c, axc, ayc,
             pmax_v, pidx_v, psel_v, t10_v, t10b, hbuf, pbuf, trow_v,
             rowcL, rowcA, sortbuf, merge_v, outbuf, shared, sem):
    c = lax.axis_index("c")
    s = lax.axis_index("s")
    b = c * 8 + s // 2
    half = s % 2
    m_base = half * MH

    iota = lax.iota(jnp.int32, L)
    iotaf = iota.astype(jnp.float32)
    neg1 = jnp.full((L,), -1.0, jnp.float32)
    zf = jnp.zeros((L,), jnp.float32)
    ones = jnp.full((L,), 1.0, jnp.float32)
    negbig = jnp.full((L,), -3.0e38, jnp.float32)
    lane0 = iota == 0

    bf = lax.convert_element_type(b, jnp.float32)

    HL = MH * L
    for k in range(7):
        pltpu.sync_copy(
            gtall.at[pl.ds(b * 7 * SL + k * SL + m_base * L, HL)],
            rowcL.at[pl.ds(k * HL, HL)])
    for k in range(5):
        pltpu.sync_copy(gtall.at[pl.ds(b * 7 * SL + k * SL, SL)],
                        rowcA.at[pl.ds(k * SL, SL)])
    pltpu.sync_copy(gtall.at[pl.ds(b * 7 * SL + 6 * SL, SL)],
                    rowcA.at[pl.ds(5 * SL, SL)])

    def ga_one(m, _):
        g1 = rowcL[pl.ds(1 * HL + m * L, L)]
        h1 = rowcL[pl.ds(2 * HL + m * L, L)]
        g2 = rowcL[pl.ds(3 * HL + m * L, L)]
        h2 = rowcL[pl.ds(4 * HL + m * L, L)]
        rowcL[pl.ds(7 * HL + m * L, L)] = (g2 - g1) * (h2 - h1)
        return 0
    lax.fori_loop(0, MH, ga_one, 0)

    def init_anchor(i, _):
        sl = pl.ds(i * L, L)
        pmax_v[sl] = neg1
        pidx_v[sl] = zf
        psel_v[sl] = zf
        return 0
    lax.fori_loop(0, PN // L, init_anchor, 0)

    def init_t10(i, _):
        t10_v[pl.ds(i * L, L)] = neg1
        return 0
    lax.fori_loop(0, MH * TOPK, init_t10, 0)
    hbuf[pl.ds(L, L)] = negbig
    pbuf[pl.ds(0, L)] = zf

    cvec = jnp.full((L,), float(C), jnp.float32)
    nchv = jnp.full((L,), float(NCHUNK), jnp.float32)
    for g in range(MH // L):
        acc = zf
        for j in range(L):
            lblj = rowcL[pl.ds((g * L + j) * L, L)]
            acc = jnp.where(iotaf == float(j), lblj, acc)
        basebuf[pl.ds(g * L, L)] = (bf * cvec + acc) * nchv

    def stage_chunk(ck):
        ckf = lax.convert_element_type(ck, jnp.float32)
        for g in range(MH // L):
            basef = basebuf[pl.ds(g * L, L)]
            cidx[pl.ds(g * L, L)] = (basef + ones * ckf).astype(jnp.int32)
        pltpu.async_copy(srows.at[cidx], rows_v, sem).wait()
        base = b * 4 * NP + ck * NK
        pltpu.sync_copy(pdtflat.at[pl.ds(base + 0 * NP, NK)], x1c)
        pltpu.sync_copy(pdtflat.at[pl.ds(base + 1 * NP, NK)], y1c)
        pltpu.sync_copy(pdtflat.at[pl.ds(base + 2 * NP, NK)], x2c)
        pltpu.sync_copy(pdtflat.at[pl.ds(base + 3 * NP, NK)], y2c)
        pltpu.sync_copy(ancflat.at[pl.ds(0 * NP + ck * NK, NK)], axc)
        pltpu.sync_copy(ancflat.at[pl.ds(1 * NP + ck * NK, NK)], ayc)

    def load_anchor(v):
        sl = pl.ds(v * L, L)
        x1 = x1c[sl]
        y1 = y1c[sl]
        x2 = x2c[sl]
        y2 = y2c[sl]
        return (x1, y1, x2, y2, axc[sl], ayc[sl],
                (x2 - x1) * (y2 - y1))

    def align_vec(v, m, av):
        x1, y1, x2, y2, ax, ay, pa = av
        HL = MH * L
        g1 = rowcL[pl.ds(1 * HL + m * L, L)]
        h1 = rowcL[pl.ds(2 * HL + m * L, L)]
        g2 = rowcL[pl.ds(3 * HL + m * L, L)]
        h2 = rowcL[pl.ds(4 * HL + m * L, L)]
        ga = rowcL[pl.ds(7 * HL + m * L, L)]
        sc_raw = rows_v[m, pl.ds(v * L, L)]
        sig = 1.0 / (1.0 + jnp.exp(-sc_raw))
        iw = jnp.maximum(jnp.minimum(x2, g2) - jnp.maximum(x1, g1), 0.0)
        ih = jnp.maximum(jnp.minimum(y2, h2) - jnp.maximum(y1, h1), 0.0)
        inter = iw * ih
        union = pa + ga - inter + 1e-7
        iou = inter / union
        iou2 = iou * iou
        iou6 = iou2 * iou2 * iou2
        ing = ((ax >= g1) & (ax <= g2)) & ((ay >= h1) & (ay <= h2))
        return jnp.where(ing, sig * iou6, 0.0)

    def p1_chunk(ck, _):
        stage_chunk(ck)

        def p1_vec(v, _):
            av = load_anchor(v)
            off = ck * NK + v * L

            def p1_row(m, _):
                al = align_vec(v, m, av)
                x = al
                for gg in range(TOPK):
                    sl10 = pl.ds((m * TOPK + gg) * L, L)
                    tgv = t10_v[sl10]
                    t10_v[sl10] = jnp.maximum(tgv, x)
                    x = jnp.minimum(tgv, x)

                pm = pmax_v[pl.ds(off, L)]
                better = al > pm
                pmax_v[pl.ds(off, L)] = jnp.where(better, al, pm)
                pi = pidx_v[pl.ds(off, L)]
                mval = rowcL[pl.ds(6 * MH * L + m * L, L)]
                pidx_v[pl.ds(off, L)] = jnp.where(better, mval, pi)
                return 0

            lax.fori_loop(0, MH, p1_row, 0)
            return 0

        lax.fori_loop(0, VPC, p1_vec, 0)
        return 0

    lax.fori_loop(0, NCHUNK, p1_chunk, 0)

    st = MH * L
    topkf = jnp.full((L,), float(TOPK), jnp.float32)

    def splat0():
        for k in range(1, L):
            hbuf[pl.ds(k, L)] = hbuf[pl.ds(k - 1, L)]
        return hbuf[pl.ds(0, L)]

    def hmax_splat():
        hbuf[pl.ds(L, L)] = negbig
        for step in (8, 4, 2, 1):
            y = jnp.maximum(hbuf[pl.ds(0, L)], hbuf[pl.ds(step, L)])
            hbuf[pl.ds(0, L)] = y
        return splat0()

    def total_splat(rank):
        hbuf[pl.ds(0, L)] = rank
        hbuf[pl.ds(L, L)] = zf
        hbuf[pl.ds(0, L)] = hbuf[pl.ds(L - 1, L)]
        return splat0()

    def prefix_incl(xf):
        pbuf[pl.ds(L, L)] = xf
        for r in (1, 2, 4, 8):
            pbuf[pl.ds(L, L)] = (pbuf[pl.ds(L, L)]
                                 + pbuf[pl.ds(L - r, L)])
        return pbuf[pl.ds(L, L)]

    def row_thresh(m, _):
        for gg in range(TOPK):
            t10b[pl.ds(gg * L, L)] = t10_v[pl.ds((m * TOPK + gg) * L, L)]

        def one_round(r, _c):
            acc = negbig
            for gg in range(TOPK):
                acc = jnp.maximum(acc, t10_v[pl.ds((m * TOPK + gg) * L, L)])
            hbuf[pl.ds(0, L)] = acc
            tv = hmax_splat()
            sortbuf[pl.ds(0, L)] = tv
            tv = sortbuf[pl.ds(0, L)]
            carry = zf
            for gg in range(TOPK):
                sl10 = pl.ds((m * TOPK + gg) * L, L)
                cur = t10_v[sl10]
                eq = cur == tv
                rank = prefix_incl(jnp.where(eq, 1.0, 0.0))
                kill = eq & ((rank + carry) == 1.0)
                t10_v[sl10] = jnp.where(kill, negbig, cur)
                carry = carry + total_splat(rank)
            return 0

        lax.fori_loop(0, TOPK, one_round, 0)
        tv = sortbuf[pl.ds(0, L)]
        gacc = zf
        for gg in range(TOPK):
            gacc = gacc + jnp.where(t10b[pl.ds(gg * L, L)] > tv, 1.0, 0.0)
        g = total_splat(prefix_incl(gacc))
        trow_v[pl.ds(m * L, L)] = tv
        trow_v[pl.ds(st + m * L, L)] = topkf - g
        trow_v[pl.ds(2 * st + m * L, L)] = zf
        return 0
    lax.fori_loop(0, MH, row_thresh, 0)

    def p2_chunk(ck, _):
        stage_chunk(ck)

        def p2_vec(v, _):
            off = ck * NK + v * L
            pidx = pidx_v[pl.ds(off, L)]
            av = load_anchor(v)

            def p2_row(m, sel_acc):
                al = align_vec(v, m, av)
                tv = trow_v[pl.ds(m * L, L)]
                need = trow_v[pl.ds(st + m * L, L)]
                carry = trow_v[pl.ds(2 * st + m * L, L)]
                gtm = al > tv
                eq = al == tv
                rank = prefix_incl(jnp.where(eq, 1.0, 0.0))
                sel_eq = eq & ((rank + carry) <= need)
                trow_v[pl.ds(2 * st + m * L, L)] = (
                    carry + total_splat(rank))
                sel = gtm | sel_eq
                mgb = rowcL[pl.ds(5 * MH * L + m * L, L)]
                mval = rowcL[pl.ds(6 * MH * L + m * L, L)]
                hit = sel & (pidx == mval)
                return jnp.where(hit, mgb, sel_acc)

            psel = lax.fori_loop(0, MH, p2_row, psel_v[pl.ds(off, L)])
            psel_v[pl.ds(off, L)] = psel
            return 0

        lax.fori_loop(0, VPC, p2_vec, 0)
        return 0

    lax.fori_loop(0, NCHUNK, p2_chunk, 0)

    pltpu.sync_copy(pmax_v.at[pl.ds(0, N)],
                    shared.at[pl.ds((s * 3 + 0) * N, N)])
    pltpu.sync_copy(pidx_v.at[pl.ds(0, N)],
                    shared.at[pl.ds((s * 3 + 1) * N, N)])
    pltpu.sync_copy(psel_v.at[pl.ds(0, N)],
                    shared.at[pl.ds((s * 3 + 2) * N, N)])
    plsc.subcore_barrier()

    other = jnp.where(half == 0, s + 1, s - 1)

    def emit_half(lo, hn, my0):
        for k in range(3):
            pltpu.sync_copy(
                shared.at[pl.ds((other * 3 + k) * N + my0, hn)],
                merge_v.at[pl.ds(k * MRG, hn)])

        def out_vec(v, _):
            na = my0 + v * L
            amax = pmax_v[pl.ds(na, L)]
            aidx = pidx_v[pl.ds(na, L)]
            asel = psel_v[pl.ds(na, L)]
            omax = merge_v[pl.ds(0 * MRG + v * L, L)]
            oidx = merge_v[pl.ds(1 * MRG + v * L, L)]
            osel = merge_v[pl.ds(2 * MRG + v * L, L)]
            use_mine = (amax >= omax) if lo else (amax > omax)
            widx = jnp.where(use_mine, aidx, oidx)
            wsel = jnp.where(use_mine, asel, osel)
            pos = wsel > 0.0
            tgf = jnp.where(pos, widx, 0.0)

            def gath(m, acc):
                l_a, x1_a, y1_a, x2_a, y2_a = acc
                moff = m * L
                hit = tgf == rowcA[pl.ds(5 * SL + moff, L)]
                l_a = jnp.where(hit, rowcA[pl.ds(0 * SL + moff, L)], l_a)
                x1_a = jnp.where(hit, rowcA[pl.ds(1 * SL + moff, L)], x1_a)
                y1_a = jnp.where(hit, rowcA[pl.ds(2 * SL + moff, L)], y1_a)
                x2_a = jnp.where(hit, rowcA[pl.ds(3 * SL + moff, L)], x2_a)
                y2_a = jnp.where(hit, rowcA[pl.ds(4 * SL + moff, L)], y2_a)
                return (l_a, x1_a, y1_a, x2_a, y2_a)

            lbl, bx1, by1, bx2, by2 = lax.fori_loop(
                0, M, gath, (zf, zf, zf, zf, zf))
            outbuf[pl.ds(0 * MRG + v * L, L)] = lbl * wsel
            outbuf[pl.ds(1 * MRG + v * L, L)] = wsel
            outbuf[pl.ds(2 * MRG + v * L, L)] = tgf
            outbuf[pl.ds(3 * MRG + v * L, L)] = bx1 * wsel
            outbuf[pl.ds(4 * MRG + v * L, L)] = by1 * wsel
            outbuf[pl.ds(5 * MRG + v * L, L)] = bx2 * wsel
            outbuf[pl.ds(6 * MRG + v * L, L)] = by2 * wsel
            return 0

        lax.fori_loop(0, (hn + L - 1) // L, out_vec, 0)
        for j in range(7):
            pltpu.sync_copy(outbuf.at[pl.ds(j * MRG, hn)],
                            out_o.at[pl.ds(j * B * N + b * N + my0, hn)])

    @pl.when(half == 0)
    def _():
        emit_half(True, HN0, 0)

    @pl.when(half == 1)
    def _():
        emit_half(False, HN1, HN0)


def _sc_call(srows, pdtflat, ancflat, gtall):
    mesh = plsc.VectorSubcoreMesh(core_axis_name="c", subcore_axis_name="s",
                                  num_cores=2, num_subcores=16)
    f = pl.kernel(
        _sc_body,
        out_type=[jax.ShapeDtypeStruct((7 * B * N,), jnp.float32)],
        mesh=mesh,
        scratch_types=[
            pltpu.VMEM((MH, NK), jnp.float32),
            pltpu.VMEM((MH,), jnp.float32),
            pltpu.VMEM((MH,), jnp.int32),
            pltpu.VMEM((NK,), jnp.float32),
            pltpu.VMEM((NK,), jnp.float32),
            pltpu.VMEM((NK,), jnp.float32),
            pltpu.VMEM((NK,), jnp.float32),
            pltpu.VMEM((NK,), jnp.float32),
            pltpu.VMEM((NK,), jnp.float32),
            pltpu.VMEM((PN,), jnp.float32),
            pltpu.VMEM((PN,), jnp.float32),
            pltpu.VMEM((PN,), jnp.float32),
            pltpu.VMEM((MH * TOPK * L,), jnp.float32),
            pltpu.VMEM((TOPK * L,), jnp.float32),
            pltpu.VMEM((2 * L,), jnp.float32),
            pltpu.VMEM((2 * L,), jnp.float32),
            pltpu.VMEM((3 * MH * L,), jnp.float32),
            pltpu.VMEM((8 * MH * L,), jnp.float32),
            pltpu.VMEM((6 * SL,), jnp.float32),
            pltpu.VMEM((4 * L,), jnp.float32),
            pltpu.VMEM((3 * MRG,), jnp.float32),
            pltpu.VMEM((7 * MRG,), jnp.float32),
            pltpu.VMEM_SHARED((16 * 3 * N,), jnp.float32),
            pltpu.SemaphoreType.DMA,
        ],
    )
    return f(srows, pdtflat, ancflat, gtall)


def kernel(pd_scores, pd_bboxes, anc_points, gt_labels, gt_bboxes, mask_gt):
    pad = ((0, 0), (0, 0), (0, NP - N))
    srows = jnp.pad(jnp.transpose(pd_scores, (0, 2, 1)),
                    pad).reshape(B * C * NCHUNK, NK)
    pdtflat = jnp.pad(jnp.transpose(pd_bboxes, (0, 2, 1)),
                      pad).reshape(-1)
    ancflat = jnp.pad(jnp.transpose(anc_points, (1, 0)),
                      ((0, 0), (0, NP - N))).reshape(-1)

    glf = gt_labels[..., 0].astype(jnp.float32)
    mgf = mask_gt[..., 0]
    mval = jnp.broadcast_to(jnp.arange(M, dtype=jnp.float32)[None], (B, M))
    secs = [glf, gt_bboxes[..., 0], gt_bboxes[..., 1],
            gt_bboxes[..., 2], gt_bboxes[..., 3], mgf, mval]
    gtall = jnp.stack(secs, axis=1)
    gtall = jnp.repeat(gtall.reshape(B * 7 * M), L)

    flat = _sc_call(srows, pdtflat, ancflat, gtall)[0]
    o = flat.reshape(7, B, N)
    tl, mp, tgf = o[0], o[1], o[2]
    tb = jnp.stack([o[3], o[4], o[5], o[6]], axis=-1)
    return (tl, tb, mp, tgf.astype(jnp.int32))
